# baseline (speedup 1.0000x reference)
import functools

import jax
import jax.numpy as jnp
from jax import lax
from jax.experimental import pallas as pl
from jax.experimental.pallas import tpu as pltpu
from jax.experimental.pallas import tpu_sc as plsc

_CHUNK = 512
_NBUF = 4
_LAG = 3


@functools.cache
def _build(n_rows, dim, chunk, nbuf, lag):
    mesh = plsc.VectorSubcoreMesh(core_axis_name="c", subcore_axis_name="s")
    nc = mesh.num_cores
    ns = mesh.num_subcores
    n_workers = nc * ns
    rows_per_w = n_rows // n_workers
    n_chunks = rows_per_w // chunk

    def body(idx_hbm, table_hbm, out_hbm, idx_v, rows_v, gsems, wsems):
        wid = lax.axis_index("s") * nc + lax.axis_index("c")
        base = wid * rows_per_w
        pltpu.sync_copy(idx_hbm.at[pl.ds(base, rows_per_w)], idx_v)

        gh = [None] * nbuf
        wh = [None] * nbuf

        def start_gather(ci):
            b = ci % nbuf
            gh[b] = pltpu.async_copy(
                table_hbm.at[idx_v.at[pl.ds(ci * chunk, chunk)]],
                rows_v.at[b],
                gsems[b],
            )

        def start_write(ci):
            b = ci % nbuf
            wh[b] = pltpu.async_copy(
                rows_v.at[b],
                out_hbm.at[pl.ds(base + ci * chunk, chunk)],
                wsems[b],
            )

        for i in range(n_chunks + lag):
            if i < n_chunks:
                b = i % nbuf
                if i >= nbuf:
                    wh[b].wait()
                start_gather(i)
            j = i - lag
            if 0 <= j < n_chunks:
                gh[j % nbuf].wait()
                start_write(j)
        for j in range(max(n_chunks - nbuf, 0), n_chunks):
            wh[j % nbuf].wait()

    return pl.kernel(
        body,
        out_type=jax.ShapeDtypeStruct((n_rows, dim), jnp.float32),
        mesh=mesh,
        scratch_types=[
            pltpu.VMEM((rows_per_w,), jnp.int32),
            pltpu.VMEM((nbuf, chunk, dim), jnp.float32),
            [pltpu.SemaphoreType.DMA] * nbuf,
            [pltpu.SemaphoreType.DMA] * nbuf,
        ],
        compiler_params=pltpu.CompilerParams(use_tc_tiling_on_sc=False),
    )


def kernel(indices, table):
    b, h = indices.shape
    _, d = table.shape
    n_rows = b * h
    idx_flat = indices.reshape(n_rows).astype(jnp.int32)
    out = _build(n_rows, d, _CHUNK, _NBUF, _LAG)(idx_flat, table)
    return out.reshape(b, h, d)
